# Initial kernel scaffold; baseline (speedup 1.0000x reference)
import jax
import jax.numpy as jnp
from jax.experimental import pallas as pl


def kernel(x, edge_index, batch, emb, W1, a1s, a1d, b1, W2, a2s, a2d, b2, W3, a3s, a3d, b3, W4, a4s, a4d, b4):
    raise NotImplementedError("write your pallas kernel here")


# speedup vs baseline: 56.6100x; 56.6100x over previous
import functools

import jax
import jax.numpy as jnp
from jax import lax
from jax.experimental import pallas as pl
from jax.experimental.pallas import tpu as pltpu
from jax.experimental.pallas import tpu_sc as plsc

F32 = jnp.float32
I32 = jnp.int32

NC = 2
NS = 16
NW = NC * NS
CH = 2048
R = 512
EPS = 1e-16


def _cdiv(a, b):
  return (a + b - 1) // b


def _make_edge_logits(H, n_pad, E_pad, nchunk):
  mesh = plsc.VectorSubcoreMesh(core_axis_name="c", subcore_axis_name="s")
  rpt = n_pad // NS
  half = rpt // 2

  out_type = (
      jax.ShapeDtypeStruct((E_pad, H), F32),
      jax.ShapeDtypeStruct((2 * n_pad, H), F32),
  )
  scratch = [
      pltpu.VMEM((16, 128), I32),
      pltpu.VMEM((16, 128), I32),
      pltpu.VMEM((CH, 2 * H), F32),
      pltpu.VMEM((CH, 2 * H), F32),
      pltpu.VMEM((CH, H), F32),
      pltpu.VMEM_SHARED((n_pad, H), F32),
      pltpu.SemaphoreType.DMA,
  ]

  @functools.partial(pl.kernel, out_type=out_type, mesh=mesh,
                     scratch_types=scratch,
                     compiler_params=pltpu.CompilerParams(
                         needs_layout_passes=False,
                         use_tc_tiling_on_sc=False))
  def kern(src2d, dst2d, alsd, z_h, ex_out, s_out,
           sidx, didx, asrc, adst, exb, s_acc, sem):
    cid = lax.axis_index("c")
    sid = lax.axis_index("s")
    wid = sid * NC + cid

    pltpu.sync_copy(z_h, s_acc.at[pl.ds(sid * rpt, rpt)])
    plsc.subcore_barrier()

    def chunk(k, _):
      roff = (wid * nchunk + k) * (CH // 128)
      pltpu.sync_copy(src2d.at[pl.ds(roff, CH // 128)], sidx)
      pltpu.sync_copy(dst2d.at[pl.ds(roff, CH // 128)], didx)

      def fire(j, _):
        pltpu.async_copy(alsd.at[sidx.at[j]], asrc.at[pl.ds(j * 128, 128)],
                         sem)
        pltpu.async_copy(alsd.at[didx.at[j]], adst.at[pl.ds(j * 128, 128)],
                         sem)
        return 0
      lax.fori_loop(0, CH // 128, fire, 0)

      def drain(j, _):
        pltpu.make_async_copy(alsd.at[sidx.at[j]],
                              asrc.at[pl.ds(j * 128, 128)], sem).wait()
        pltpu.make_async_copy(alsd.at[didx.at[j]],
                              adst.at[pl.ds(j * 128, 128)], sem).wait()
        return 0
      lax.fori_loop(0, CH // 128, drain, 0)

      def comp(g, _):
        rows = g * 16 + lax.iota(I32, 16)
        for hd in range(H):
          hvec = jnp.full((16,), hd, I32)
          a_s = plsc.load_gather(asrc, [rows, hvec])
          a_d = plsc.load_gather(adst, [rows, hvec + H])
          e = a_s + a_d
          e = jnp.where(e >= 0.0, e, 0.2 * e)
          plsc.store_scatter(exb, [rows, hvec], jnp.exp(e))
        return 0
      lax.fori_loop(0, CH // 16, comp, 0, unroll=2)

      pltpu.sync_copy(exb, ex_out.at[pl.ds(roff * 128, CH)])

      def scat(j, _):
        pltpu.sync_copy(exb.at[pl.ds(j * 128, 128)], s_acc.at[didx.at[j]],
                        add=True)
        return 0
      lax.fori_loop(0, CH // 128, scat, 0)
      return 0

    lax.fori_loop(0, nchunk, chunk, 0)
    plsc.subcore_barrier()

    base = sid * rpt
    for t in range(2):
      pltpu.sync_copy(s_acc.at[pl.ds(base + t * half, half)],
                      exb.at[pl.ds(0, half)])
      pltpu.sync_copy(exb.at[pl.ds(0, half)],
                      s_out.at[pl.ds(cid * n_pad + base + t * half, half)])

  return kern


def _make_edge_aggregate(H, D, n_pad, E_pad, nchunk, ch):
  mesh = plsc.VectorSubcoreMesh(core_axis_name="c", subcore_axis_name="s")
  rpt = n_pad // NS
  nhop = 8
  hopsz = rpt // nhop
  assert hopsz <= ch

  out_type = jax.ShapeDtypeStruct((2 * H * n_pad, D), F32)
  scratch = [
      pltpu.VMEM((ch // 128, 128), I32),
      pltpu.VMEM((ch // 128, 128), I32),
      pltpu.VMEM((ch, H), F32),
      pltpu.VMEM((ch, D), F32),
      pltpu.VMEM_SHARED((n_pad, D), F32),
      pltpu.SemaphoreType.DMA,
  ]

  @functools.partial(pl.kernel, out_type=out_type, mesh=mesh,
                     scratch_types=scratch,
                     compiler_params=pltpu.CompilerParams(
                         needs_layout_passes=False,
                         use_tc_tiling_on_sc=False))
  def kern(src2d, dst2d, ex_in, h_t, z_d, u_out,
           sidx, didx, exb, hrows, u_acc, sem):
    cid = lax.axis_index("c")
    sid = lax.axis_index("s")
    wid = sid * NC + cid

    for hd in range(H):
      table = h_t.at[hd]
      pltpu.sync_copy(z_d, u_acc.at[pl.ds(sid * rpt, rpt)])
      plsc.subcore_barrier()

      def chunk(k, _):
        roff = (wid * nchunk + k) * (ch // 128)
        pltpu.sync_copy(src2d.at[pl.ds(roff, ch // 128)], sidx)
        pltpu.sync_copy(dst2d.at[pl.ds(roff, ch // 128)], didx)
        pltpu.sync_copy(ex_in.at[pl.ds(roff * 128, ch)], exb)

        def fire(j, _):
          pltpu.async_copy(table.at[sidx.at[j]],
                           hrows.at[pl.ds(j * 128, 128)], sem)
          return 0
        lax.fori_loop(0, ch // 128, fire, 0)

        def drain(j, _):
          pltpu.make_async_copy(table.at[sidx.at[j]],
                                hrows.at[pl.ds(j * 128, 128)], sem).wait()
          return 0
        lax.fori_loop(0, ch // 128, drain, 0)

        def comp(g, _):
          rows = g * 16 + lax.iota(I32, 16)
          hvec = jnp.full((16,), hd, I32)
          ex16 = plsc.load_gather(exb, [rows, hvec])
          base = g * 16
          for j in range(16):
            xi = ex16[j]
            for c0 in range(0, D, 16):
              hrows[base + j, pl.ds(c0, 16)] = (
                  hrows[base + j, pl.ds(c0, 16)] * xi)
          return 0
        lax.fori_loop(0, ch // 16, comp, 0)

        def scat(j, _):
          pltpu.sync_copy(hrows.at[pl.ds(j * 128, 128)],
                          u_acc.at[didx.at[j]], add=True)
          return 0
        lax.fori_loop(0, ch // 128, scat, 0)
        return 0

      lax.fori_loop(0, nchunk, chunk, 0)
      plsc.subcore_barrier()

      base = sid * rpt
      urow = (cid * H + hd) * n_pad
      for t in range(nhop):
        pltpu.sync_copy(u_acc.at[pl.ds(base + t * hopsz, hopsz)],
                        hrows.at[pl.ds(0, hopsz)])
        pltpu.sync_copy(hrows.at[pl.ds(0, hopsz)],
                        u_out.at[pl.ds(urow + base + t * hopsz, hopsz)])

  return kern


def _tc_layer1(n_pad, num_ids, emb_dim, h_out, heads):
  d = h_out // heads
  grid = n_pad // R

  def body(x_ref, emb_ref, w_ref, a_ref, ht_ref, alsd_ref):
    xb = x_ref[...]
    ids = xb[:, 0].astype(I32)
    col = lax.broadcasted_iota(I32, (R, num_ids), 1)
    onehot = (col == ids[:, None]).astype(F32)
    eblk = jnp.dot(onehot, emb_ref[...], preferred_element_type=F32)
    pad = jnp.zeros((R, 1), F32)
    hin = jnp.concatenate([eblk, xb[:, 1:], pad], axis=1)
    h = jnp.dot(hin, w_ref[...], preferred_element_type=F32)
    alsd_ref[...] = jnp.dot(h, a_ref[...], preferred_element_type=F32)
    for hd in range(heads):
      ht_ref[hd] = h[:, hd * d:(hd + 1) * d]

  return pl.pallas_call(
      body,
      grid=(grid,),
      in_specs=[
          pl.BlockSpec((R, 16), lambda i: (i, 0)),
          pl.BlockSpec((num_ids, emb_dim), lambda i: (0, 0)),
          pl.BlockSpec((24, h_out), lambda i: (0, 0)),
          pl.BlockSpec((h_out, 2 * heads), lambda i: (0, 0)),
      ],
      out_specs=[
          pl.BlockSpec((heads, R, d), lambda i: (0, i, 0)),
          pl.BlockSpec((R, 2 * heads), lambda i: (i, 0)),
      ],
      out_shape=[
          jax.ShapeDtypeStruct((heads, n_pad, d), F32),
          jax.ShapeDtypeStruct((n_pad, 2 * heads), F32),
      ],
  )


def _tc_mid(n_pad, hp, dp, f_out, heads):
  d = f_out // heads
  grid = n_pad // R

  def body(u_ref, s_ref, b_ref, w_ref, a_ref, ht_ref, alsd_ref):
    s_sum = s_ref[0] + s_ref[1]
    blocks = []
    for hd in range(hp):
      u = u_ref[0, hd] + u_ref[1, hd]
      blocks.append(u / (s_sum[:, hd:hd + 1] + EPS))
    hin = jnp.concatenate(blocks, axis=1) if hp > 1 else blocks[0]
    hin = jnp.maximum(hin + b_ref[0], 0.0)
    h = jnp.dot(hin, w_ref[...], preferred_element_type=F32)
    alsd_ref[...] = jnp.dot(h, a_ref[...], preferred_element_type=F32)
    for hd in range(heads):
      ht_ref[hd] = h[:, hd * d:(hd + 1) * d]

  return pl.pallas_call(
      body,
      grid=(grid,),
      in_specs=[
          pl.BlockSpec((2, hp, R, dp), lambda i: (0, 0, i, 0)),
          pl.BlockSpec((2, R, hp), lambda i: (0, i, 0)),
          pl.BlockSpec((1, hp * dp), lambda i: (0, 0)),
          pl.BlockSpec((hp * dp, f_out), lambda i: (0, 0)),
          pl.BlockSpec((f_out, 2 * heads), lambda i: (0, 0)),
      ],
      out_specs=[
          pl.BlockSpec((heads, R, d), lambda i: (0, i, 0)),
          pl.BlockSpec((R, 2 * heads), lambda i: (i, 0)),
      ],
      out_shape=[
          jax.ShapeDtypeStruct((heads, n_pad, d), F32),
          jax.ShapeDtypeStruct((n_pad, 2 * heads), F32),
      ],
  )


def _tc_final(n_pad, dp):
  grid = n_pad // R

  def body(u_ref, s_ref, b_ref, o_ref):
    s_sum = s_ref[0] + s_ref[1]
    u = u_ref[0, 0] + u_ref[1, 0]
    o_ref[...] = jax.nn.sigmoid(u / (s_sum + EPS) + b_ref[0])

  return pl.pallas_call(
      body,
      grid=(grid,),
      in_specs=[
          pl.BlockSpec((2, 1, R, dp), lambda i: (0, 0, i, 0)),
          pl.BlockSpec((2, R, 1), lambda i: (0, i, 0)),
          pl.BlockSpec((1, dp), lambda i: (0, 0)),
      ],
      out_specs=pl.BlockSpec((R, dp), lambda i: (i, 0)),
      out_shape=jax.ShapeDtypeStruct((n_pad, dp), F32),
  )


def _combine_attn(a_s, a_d):
  heads, d = a_s.shape
  eye = jnp.eye(heads, dtype=F32)
  blk_s = (a_s[:, :, None] * eye[:, None, :]).reshape(heads * d, heads)
  blk_d = (a_d[:, :, None] * eye[:, None, :]).reshape(heads * d, heads)
  return jnp.concatenate([blk_s, blk_d], axis=1)


def kernel(x, edge_index, batch, emb, W1, a1s, a1d, b1, W2, a2s, a2d, b2,
           W3, a3s, a3d, b3, W4, a4s, a4d, b4):
  n = x.shape[0]
  e_raw = edge_index.shape[1]
  e_tot = e_raw + n
  num_ids, emb_dim = emb.shape

  n_pad = _cdiv(n, R) * R
  per_worker = _cdiv(e_tot, NW)
  nchunk = _cdiv(per_worker, CH)
  e_pad = NW * nchunk * CH

  loop = jnp.arange(n, dtype=I32)
  src = jnp.concatenate([edge_index[0].astype(I32), loop])
  dst = jnp.concatenate([edge_index[1].astype(I32), loop])
  dummy = jnp.full((e_pad - e_tot,), n, I32)
  src2d = jnp.concatenate([src, dummy]).reshape(e_pad // 128, 128)
  dst2d = jnp.concatenate([dst, dummy]).reshape(e_pad // 128, 128)

  x_pad = jnp.pad(x, ((0, n_pad - n), (0, 0)))
  w1p = jnp.pad(W1, ((0, 24 - W1.shape[0]), (0, 0)))
  A1, A2 = _combine_attn(a1s, a1d), _combine_attn(a2s, a2d)
  A3, A4 = _combine_attn(a3s, a3d), _combine_attn(a4s, a4d)

  rpt = n_pad // NS
  z1 = jnp.zeros((rpt, 1), F32)
  z4 = jnp.zeros((rpt, 4), F32)
  z16 = jnp.zeros((rpt, 16), F32)
  z32 = jnp.zeros((rpt, 32), F32)

  def run_layer(h_t, alsd, heads, d):
    ch = 512 if d == 32 else 1024
    ex, s_flat = _make_edge_logits(heads, n_pad, e_pad, nchunk)(
        src2d, dst2d, alsd, z4 if heads == 4 else z1)
    u_flat = _make_edge_aggregate(heads, d, n_pad, e_pad, nchunk * CH // ch,
                                  ch)(
        src2d, dst2d, ex, h_t, z32 if d == 32 else z16)
    u = u_flat.reshape(2, heads, n_pad, d)
    s = s_flat.reshape(2, n_pad, heads)
    return u, s

  h_t, alsd = _tc_layer1(n_pad, num_ids, emb_dim, 128, 4)(
      x_pad, emb, w1p, A1)
  u, s = run_layer(h_t, alsd, 4, 32)
  h_t, alsd = _tc_mid(n_pad, 4, 32, 32, 1)(u, s, b1[None, :], W2, A2)
  u, s = run_layer(h_t, alsd, 1, 32)
  h_t, alsd = _tc_mid(n_pad, 1, 32, 128, 4)(u, s, b2[None, :], W3, A3)
  u, s = run_layer(h_t, alsd, 4, 32)
  h_t, alsd = _tc_mid(n_pad, 4, 32, 16, 1)(u, s, b3[None, :], W4, A4)
  u, s = run_layer(h_t, alsd, 1, 16)
  out = _tc_final(n_pad, 16)(u, s, b4[None, :])
  return out[:n]


# speedup vs baseline: 76.3143x; 1.3481x over previous
import functools

import jax
import jax.numpy as jnp
from jax import lax
from jax.experimental import pallas as pl
from jax.experimental.pallas import tpu as pltpu
from jax.experimental.pallas import tpu_sc as plsc

F32 = jnp.float32
I32 = jnp.int32

NC = 2
NS = 16
NW = NC * NS
CH = 1536
R = 512
EPS = 1e-16


def _cdiv(a, b):
  return (a + b - 1) // b


def _make_edge_logits(H, n_pad, E_pad, nchunk):
  mesh = plsc.VectorSubcoreMesh(core_axis_name="c", subcore_axis_name="s")
  rpt = n_pad // NS
  qtr = rpt // 4

  out_type = (
      jax.ShapeDtypeStruct((E_pad, H), F32),
      jax.ShapeDtypeStruct((2 * n_pad, H), F32),
  )
  scratch = [
      pltpu.VMEM((CH // 128, 128), I32),
      pltpu.VMEM((CH // 128, 128), I32),
      pltpu.VMEM((CH, 2 * H), F32),
      pltpu.VMEM((CH, 2 * H), F32),
      pltpu.VMEM((CH, H), F32),
      pltpu.VMEM_SHARED((n_pad, H), F32),
      pltpu.SemaphoreType.DMA,
  ]

  @functools.partial(pl.kernel, out_type=out_type, mesh=mesh,
                     scratch_types=scratch,
                     compiler_params=pltpu.CompilerParams(
                         needs_layout_passes=False,
                         use_tc_tiling_on_sc=False))
  def kern(src2d, dst2d, alsd, z_h, ex_out, s_out,
           sidx, didx, asrc, adst, exb, s_acc, sem):
    cid = lax.axis_index("c")
    sid = lax.axis_index("s")
    wid = sid * NC + cid

    pltpu.sync_copy(z_h, s_acc.at[pl.ds(sid * rpt, rpt)])
    plsc.subcore_barrier()

    def chunk(k, _):
      roff = (wid * nchunk + k) * (CH // 128)
      pltpu.sync_copy(src2d.at[pl.ds(roff, CH // 128)], sidx)
      pltpu.sync_copy(dst2d.at[pl.ds(roff, CH // 128)], didx)

      def fire(j, _):
        pltpu.async_copy(alsd.at[sidx.at[j]], asrc.at[pl.ds(j * 128, 128)],
                         sem)
        pltpu.async_copy(alsd.at[didx.at[j]], adst.at[pl.ds(j * 128, 128)],
                         sem)
        return 0
      lax.fori_loop(0, CH // 128, fire, 0)

      def drain(j, _):
        pltpu.make_async_copy(alsd.at[sidx.at[j]],
                              asrc.at[pl.ds(j * 128, 128)], sem).wait()
        pltpu.make_async_copy(alsd.at[didx.at[j]],
                              adst.at[pl.ds(j * 128, 128)], sem).wait()
        return 0
      lax.fori_loop(0, CH // 128, drain, 0)

      def comp(g, _):
        rows = g * 16 + lax.iota(I32, 16)
        for hd in range(H):
          hvec = jnp.full((16,), hd, I32)
          a_s = plsc.load_gather(asrc, [rows, hvec])
          a_d = plsc.load_gather(adst, [rows, hvec + H])
          e = a_s + a_d
          e = jnp.where(e >= 0.0, e, 0.2 * e)
          plsc.store_scatter(exb, [rows, hvec], jnp.exp(e))
        return 0
      lax.fori_loop(0, CH // 16, comp, 0, unroll=2)

      pltpu.sync_copy(exb, ex_out.at[pl.ds(roff * 128, CH)])

      def scat(j, _):
        pltpu.sync_copy(exb.at[pl.ds(j * 128, 128)], s_acc.at[didx.at[j]],
                        add=True)
        return 0
      lax.fori_loop(0, CH // 128, scat, 0)
      return 0

    lax.fori_loop(0, nchunk, chunk, 0)
    plsc.subcore_barrier()

    base = sid * rpt
    for t in range(4):
      pltpu.sync_copy(s_acc.at[pl.ds(base + t * qtr, qtr)],
                      exb.at[pl.ds(0, qtr)])
      pltpu.sync_copy(exb.at[pl.ds(0, qtr)],
                      s_out.at[pl.ds(cid * n_pad + base + t * qtr, qtr)])

  return kern


def _make_edge_aggregate(H, D, n_pad, E_pad, nchunk, ch):
  mesh = plsc.VectorSubcoreMesh(core_axis_name="c", subcore_axis_name="s")
  rpt = n_pad // NS
  nhop = 16
  hopsz = rpt // nhop
  assert hopsz <= ch and nchunk % 6 == 0 and ch % 128 == 0
  nf = ch // 128

  out_type = jax.ShapeDtypeStruct((2 * H * n_pad, D), F32)
  scratch = (
      [pltpu.VMEM((nf, 128), I32) for _ in range(3)]
      + [pltpu.VMEM((nf, 128), I32) for _ in range(3)]
      + [pltpu.VMEM((ch, H), F32) for _ in range(3)]
      + [pltpu.VMEM((ch, D), F32) for _ in range(2)]
      + [pltpu.VMEM_SHARED((n_pad, D), F32)]
      + [pltpu.SemaphoreType.DMA] * 6
  )

  @functools.partial(pl.kernel, out_type=out_type, mesh=mesh,
                     scratch_types=scratch,
                     compiler_params=pltpu.CompilerParams(
                         needs_layout_passes=False,
                         use_tc_tiling_on_sc=False))
  def kern(src2d, dst2d, ex_in, h_t, z_d, u_out,
           s0, s1, s2, d0, d1, d2, e0, e1, e2, hr0, hr1, u_acc,
           mi0, mi1, mi2, mg0, mg1, ms):
    cid = lax.axis_index("c")
    sid = lax.axis_index("s")
    wid = sid * NC + cid
    sidx = [s0, s1, s2]
    didx = [d0, d1, d2]
    exb = [e0, e1, e2]
    hrows = [hr0, hr1]
    semi = [mi0, mi1, mi2]
    semg = [mg0, mg1]

    def roff(j):
      return (wid * nchunk + j) * nf

    def fire_idxload(j, ls):
      pltpu.async_copy(src2d.at[pl.ds(roff(j), nf)], sidx[ls], semi[ls])
      pltpu.async_copy(dst2d.at[pl.ds(roff(j), nf)], didx[ls], semi[ls])
      pltpu.async_copy(ex_in.at[pl.ds(roff(j) * 128, ch)], exb[ls], semi[ls])

    def wait_idxload(j, ls):
      pltpu.make_async_copy(src2d.at[pl.ds(roff(j), nf)], sidx[ls],
                            semi[ls]).wait()
      pltpu.make_async_copy(dst2d.at[pl.ds(roff(j), nf)], didx[ls],
                            semi[ls]).wait()
      pltpu.make_async_copy(ex_in.at[pl.ds(roff(j) * 128, ch)], exb[ls],
                            semi[ls]).wait()

    for hd in range(H):
      table = h_t.at[hd]
      pltpu.sync_copy(z_d, u_acc.at[pl.ds(sid * rpt, rpt)])
      plsc.subcore_barrier()

      def fire_gather(ls, hs):
        for jj in range(nf):
          pltpu.async_copy(table.at[sidx[ls].at[jj]],
                           hrows[hs].at[pl.ds(jj * 128, 128)], semg[hs])

      def drain_gather(ls, hs):
        for jj in range(nf):
          pltpu.make_async_copy(table.at[sidx[ls].at[jj]],
                                hrows[hs].at[pl.ds(jj * 128, 128)],
                                semg[hs]).wait()

      def fire_scatter(ls, hs):
        for jj in range(nf):
          pltpu.async_copy(hrows[hs].at[pl.ds(jj * 128, 128)],
                           u_acc.at[didx[ls].at[jj]], ms, add=True)

      def wait_scatter(ls, hs):
        for jj in range(nf):
          pltpu.make_async_copy(hrows[hs].at[pl.ds(jj * 128, 128)],
                                u_acc.at[didx[ls].at[jj]], ms).wait()

      fire_idxload(0, 0)
      fire_idxload(1, 1)
      wait_idxload(0, 0)
      fire_gather(0, 0)

      def superphase(sp, _):
        for p in range(6):
          j = sp * 6 + p
          lj, lj1, lj2 = p % 3, (p + 1) % 3, (p + 2) % 3
          hj, hj1 = p % 2, (p + 1) % 2
          drain_gather(lj, hj)

          @pl.when(j >= 1)
          def _():
            wait_scatter(lj2, hj1)

          @pl.when(j + 2 < nchunk)
          def _():
            fire_idxload(j + 2, lj2)

          @pl.when(j + 1 < nchunk)
          def _():
            wait_idxload(j + 1, lj1)
            fire_gather(lj1, hj1)

          def comp(g, _):
            rows = g * 16 + lax.iota(I32, 16)
            hvec = jnp.full((16,), hd, I32)
            ex16 = plsc.load_gather(exb[lj], [rows, hvec])
            base = g * 16
            for jx in range(16):
              xi = ex16[jx]
              for c0 in range(0, D, 16):
                hrows[hj][base + jx, pl.ds(c0, 16)] = (
                    hrows[hj][base + jx, pl.ds(c0, 16)] * xi)
            return 0
          lax.fori_loop(0, ch // 16, comp, 0)

          fire_scatter(lj, hj)
        return 0

      lax.fori_loop(0, nchunk // 6, superphase, 0)
      wait_scatter((nchunk - 1) % 3, (nchunk - 1) % 2)

      plsc.subcore_barrier()
      base = sid * rpt
      urow = (cid * H + hd) * n_pad
      for t in range(nhop):
        pltpu.sync_copy(u_acc.at[pl.ds(base + t * hopsz, hopsz)],
                        hrows[0].at[pl.ds(0, hopsz)])
        pltpu.sync_copy(hrows[0].at[pl.ds(0, hopsz)],
                        u_out.at[pl.ds(urow + base + t * hopsz, hopsz)])

  return kern


def _tc_layer1(n_pad, num_ids, emb_dim, h_out, heads):
  d = h_out // heads
  grid = n_pad // R

  def body(x_ref, emb_ref, w_ref, a_ref, ht_ref, alsd_ref):
    xb = x_ref[...]
    ids = xb[:, 0].astype(I32)
    col = lax.broadcasted_iota(I32, (R, num_ids), 1)
    onehot = (col == ids[:, None]).astype(F32)
    eblk = jnp.dot(onehot, emb_ref[...], preferred_element_type=F32)
    pad = jnp.zeros((R, 1), F32)
    hin = jnp.concatenate([eblk, xb[:, 1:], pad], axis=1)
    h = jnp.dot(hin, w_ref[...], preferred_element_type=F32)
    alsd_ref[...] = jnp.dot(h, a_ref[...], preferred_element_type=F32)
    for hd in range(heads):
      ht_ref[hd] = h[:, hd * d:(hd + 1) * d]

  return pl.pallas_call(
      body,
      grid=(grid,),
      in_specs=[
          pl.BlockSpec((R, 16), lambda i: (i, 0)),
          pl.BlockSpec((num_ids, emb_dim), lambda i: (0, 0)),
          pl.BlockSpec((24, h_out), lambda i: (0, 0)),
          pl.BlockSpec((h_out, 2 * heads), lambda i: (0, 0)),
      ],
      out_specs=[
          pl.BlockSpec((heads, R, d), lambda i: (0, i, 0)),
          pl.BlockSpec((R, 2 * heads), lambda i: (i, 0)),
      ],
      out_shape=[
          jax.ShapeDtypeStruct((heads, n_pad, d), F32),
          jax.ShapeDtypeStruct((n_pad, 2 * heads), F32),
      ],
  )


def _tc_mid(n_pad, hp, dp, f_out, heads):
  d = f_out // heads
  grid = n_pad // R

  def body(u_ref, s_ref, b_ref, w_ref, a_ref, ht_ref, alsd_ref):
    s_sum = s_ref[0] + s_ref[1]
    blocks = []
    for hd in range(hp):
      u = u_ref[0, hd] + u_ref[1, hd]
      blocks.append(u / (s_sum[:, hd:hd + 1] + EPS))
    hin = jnp.concatenate(blocks, axis=1) if hp > 1 else blocks[0]
    hin = jnp.maximum(hin + b_ref[0], 0.0)
    h = jnp.dot(hin, w_ref[...], preferred_element_type=F32)
    alsd_ref[...] = jnp.dot(h, a_ref[...], preferred_element_type=F32)
    for hd in range(heads):
      ht_ref[hd] = h[:, hd * d:(hd + 1) * d]

  return pl.pallas_call(
      body,
      grid=(grid,),
      in_specs=[
          pl.BlockSpec((2, hp, R, dp), lambda i: (0, 0, i, 0)),
          pl.BlockSpec((2, R, hp), lambda i: (0, i, 0)),
          pl.BlockSpec((1, hp * dp), lambda i: (0, 0)),
          pl.BlockSpec((hp * dp, f_out), lambda i: (0, 0)),
          pl.BlockSpec((f_out, 2 * heads), lambda i: (0, 0)),
      ],
      out_specs=[
          pl.BlockSpec((heads, R, d), lambda i: (0, i, 0)),
          pl.BlockSpec((R, 2 * heads), lambda i: (i, 0)),
      ],
      out_shape=[
          jax.ShapeDtypeStruct((heads, n_pad, d), F32),
          jax.ShapeDtypeStruct((n_pad, 2 * heads), F32),
      ],
  )


def _tc_final(n_pad, dp):
  grid = n_pad // R

  def body(u_ref, s_ref, b_ref, o_ref):
    s_sum = s_ref[0] + s_ref[1]
    u = u_ref[0, 0] + u_ref[1, 0]
    o_ref[...] = jax.nn.sigmoid(u / (s_sum + EPS) + b_ref[0])

  return pl.pallas_call(
      body,
      grid=(grid,),
      in_specs=[
          pl.BlockSpec((2, 1, R, dp), lambda i: (0, 0, i, 0)),
          pl.BlockSpec((2, R, 1), lambda i: (0, i, 0)),
          pl.BlockSpec((1, dp), lambda i: (0, 0)),
      ],
      out_specs=pl.BlockSpec((R, dp), lambda i: (i, 0)),
      out_shape=jax.ShapeDtypeStruct((n_pad, dp), F32),
  )


def _combine_attn(a_s, a_d):
  heads, d = a_s.shape
  eye = jnp.eye(heads, dtype=F32)
  blk_s = (a_s[:, :, None] * eye[:, None, :]).reshape(heads * d, heads)
  blk_d = (a_d[:, :, None] * eye[:, None, :]).reshape(heads * d, heads)
  return jnp.concatenate([blk_s, blk_d], axis=1)


def kernel(x, edge_index, batch, emb, W1, a1s, a1d, b1, W2, a2s, a2d, b2,
           W3, a3s, a3d, b3, W4, a4s, a4d, b4):
  n = x.shape[0]
  e_raw = edge_index.shape[1]
  e_tot = e_raw + n
  num_ids, emb_dim = emb.shape

  n_pad = _cdiv(n, R) * R
  if n_pad == n:
    n_pad += R
  ew = _cdiv(e_tot, NW * 1536) * 1536
  if (ew // 256) % 6:
    ew = _cdiv(ew, 1536 * 2) * 1536 * 2
  while (ew // 256) % 6 or (ew // 512) % 6 or ew % 1536:
    ew += 1536
  nchunk = ew // 1536
  e_pad = NW * ew

  loop = jnp.arange(n, dtype=I32)
  src = jnp.concatenate([edge_index[0].astype(I32), loop])
  dst = jnp.concatenate([edge_index[1].astype(I32), loop])
  pad_rows = n_pad - n
  dummy = n + (jnp.arange(e_pad - e_tot, dtype=I32) % pad_rows)
  src2d = jnp.concatenate([src, dummy]).reshape(e_pad // 128, 128)
  dst2d = jnp.concatenate([dst, dummy]).reshape(e_pad // 128, 128)

  x_pad = jnp.pad(x, ((0, n_pad - n), (0, 0)))
  w1p = jnp.pad(W1, ((0, 24 - W1.shape[0]), (0, 0)))
  A1, A2 = _combine_attn(a1s, a1d), _combine_attn(a2s, a2d)
  A3, A4 = _combine_attn(a3s, a3d), _combine_attn(a4s, a4d)

  rpt = n_pad // NS
  z1 = jnp.zeros((rpt, 1), F32)
  z4 = jnp.zeros((rpt, 4), F32)
  z16 = jnp.zeros((rpt, 16), F32)
  z32 = jnp.zeros((rpt, 32), F32)

  def run_layer(h_t, alsd, heads, d):
    ch = 256 if d == 32 else 512
    ex, s_flat = _make_edge_logits(heads, n_pad, e_pad, nchunk)(
        src2d, dst2d, alsd, z4 if heads == 4 else z1)
    u_flat = _make_edge_aggregate(heads, d, n_pad, e_pad, ew // ch, ch)(
        src2d, dst2d, ex, h_t, z32 if d == 32 else z16)
    u = u_flat.reshape(2, heads, n_pad, d)
    s = s_flat.reshape(2, n_pad, heads)
    return u, s

  h_t, alsd = _tc_layer1(n_pad, num_ids, emb_dim, 128, 4)(
      x_pad, emb, w1p, A1)
  u, s = run_layer(h_t, alsd, 4, 32)
  h_t, alsd = _tc_mid(n_pad, 4, 32, 32, 1)(u, s, b1[None, :], W2, A2)
  u, s = run_layer(h_t, alsd, 1, 32)
  h_t, alsd = _tc_mid(n_pad, 1, 32, 128, 4)(u, s, b2[None, :], W3, A3)
  u, s = run_layer(h_t, alsd, 4, 32)
  h_t, alsd = _tc_mid(n_pad, 4, 32, 16, 1)(u, s, b3[None, :], W4, A4)
  u, s = run_layer(h_t, alsd, 1, 16)
  out = _tc_final(n_pad, 16)(u, s, b4[None, :])
  return out[:n]
